# Initial kernel scaffold; baseline (speedup 1.0000x reference)
import jax
import jax.numpy as jnp
from jax.experimental import pallas as pl


def kernel(input_x, W):
    raise NotImplementedError("write your pallas kernel here")


# speedup vs baseline: 2.8262x; 2.8262x over previous
import functools

import jax
import jax.numpy as jnp
from jax import lax
from jax.experimental import pallas as pl
from jax.experimental.pallas import tpu as pltpu
from jax.experimental.pallas import tpu_sc as plsc

NFIELD = 8
NFEAT = 26
VOCAB = 1000
D = 64
B = 1024
NPAIR = (NFEAT * (NFEAT - 1)) // 2
NROW = NFIELD * NFEAT
NC, NS = 2, 16
NW = NC * NS
BPW = B // NW
HALF = NROW // 2

_PAIRS = tuple(
    ((j % NFIELD) * NFEAT + i, (i % NFIELD) * NFEAT + j)
    for i in range(NFEAT - 1)
    for j in range(i + 1, NFEAT)
)


def _body(idx_hbm, table_hbm, out_hbm, idx_v, rows_v, out_v, sem):
    wid = lax.axis_index("s") * NC + lax.axis_index("c")
    row0 = wid * BPW
    pltpu.sync_copy(idx_hbm.at[pl.ds(row0, BPW)], idx_v)

    def one_row(r, carry):
        c0 = pltpu.async_copy(
            table_hbm.at[idx_v.at[r, 0]], rows_v.at[pl.ds(0, HALF)], sem)
        c1 = pltpu.async_copy(
            table_hbm.at[idx_v.at[r, 1]], rows_v.at[pl.ds(HALF, HALF)], sem)
        c0.wait()
        c1.wait()
        for p, (a, b) in enumerate(_PAIRS):
            for k in range(D // 16):
                s = pl.ds(16 * k, 16)
                out_v[p, s] = rows_v[a, s] * rows_v[b, s]
        pltpu.sync_copy(out_v, out_hbm.at[row0 + r])
        return carry

    lax.fori_loop(0, BPW, one_row, 0)


def kernel(input_x, W):
    token = input_x[0].astype(jnp.int32)
    f_off = jnp.arange(NFEAT, dtype=jnp.int32) * VOCAB
    g_off = jnp.arange(NFIELD, dtype=jnp.int32) * (NFEAT * VOCAB)
    idx = token[:, None, :] + f_off[None, None, :] + g_off[None, :, None]
    idx = idx.reshape(B, 2, HALF)
    table = W.reshape(NFIELD * NFEAT * VOCAB, D)

    run = pl.kernel(
        _body,
        out_type=jax.ShapeDtypeStruct((B, NPAIR, D), jnp.float32),
        mesh=plsc.VectorSubcoreMesh(
            core_axis_name="c", subcore_axis_name="s",
            num_cores=NC, num_subcores=NS),
        scratch_types=[
            pltpu.VMEM((BPW, 2, HALF), jnp.int32),
            pltpu.VMEM((NROW, D), jnp.float32),
            pltpu.VMEM((NPAIR, D), jnp.float32),
            pltpu.SemaphoreType.DMA,
        ],
        compiler_params=pltpu.CompilerParams(use_tc_tiling_on_sc=False),
    )
    return run(idx, table)


# speedup vs baseline: 3.3360x; 1.1804x over previous
import functools

import jax
import jax.numpy as jnp
from jax import lax
from jax.experimental import pallas as pl
from jax.experimental.pallas import tpu as pltpu
from jax.experimental.pallas import tpu_sc as plsc

NFIELD = 8
NFEAT = 26
VOCAB = 1000
D = 64
B = 1024
NPAIR = (NFEAT * (NFEAT - 1)) // 2
NC, NS = 2, 16
NW = NC * NS
BPW = B // NW

_PAIRS = tuple(
    (i, j, (j % NFIELD) * D, (i % NFIELD) * D)
    for i in range(NFEAT - 1)
    for j in range(i + 1, NFEAT)
)


def _body(idx_hbm, table_hbm, out_hbm, idx_v, rows_v, out_v, sem):
    wid = lax.axis_index("s") * NC + lax.axis_index("c")
    row0 = wid * BPW
    pltpu.sync_copy(idx_hbm.at[pl.ds(row0, BPW)], idx_v)

    def one_row(r, carry):
        pltpu.async_copy(table_hbm.at[idx_v.at[r]], rows_v, sem).wait()
        for p, (i, j, oa, ob) in enumerate(_PAIRS):
            for k in range(D // 16):
                s = 16 * k
                out_v[p, pl.ds(s, 16)] = (
                    rows_v[i, pl.ds(oa + s, 16)] * rows_v[j, pl.ds(ob + s, 16)])
        pltpu.sync_copy(out_v, out_hbm.at[row0 + r])
        return carry

    lax.fori_loop(0, BPW, one_row, 0)


def kernel(input_x, W):
    token = input_x[0].astype(jnp.int32)
    f_off = jnp.arange(NFEAT, dtype=jnp.int32) * VOCAB
    idx = token + f_off[None, :]
    table = jnp.transpose(W, (1, 0, 2)).reshape(NFEAT * VOCAB, NFIELD * D)

    run = pl.kernel(
        _body,
        out_type=jax.ShapeDtypeStruct((B, NPAIR, D), jnp.float32),
        mesh=plsc.VectorSubcoreMesh(
            core_axis_name="c", subcore_axis_name="s",
            num_cores=NC, num_subcores=NS),
        scratch_types=[
            pltpu.VMEM((BPW, NFEAT), jnp.int32),
            pltpu.VMEM((NFEAT, NFIELD * D), jnp.float32),
            pltpu.VMEM((NPAIR, D), jnp.float32),
            pltpu.SemaphoreType.DMA,
        ],
    )
    return run(idx, table)
